# baseline (speedup 1.0000x reference)
import functools

import jax
import jax.numpy as jnp
from jax import lax
from jax.experimental import pallas as pl
from jax.experimental.pallas import tpu as pltpu

B, S, NR, R = 2048, 8192, 2048, 512
Q, H, RK, SK = 512, 1024, 128, 128
HA = H + 128


def _mm(x, w):
    return jax.lax.dot_general(x.astype(jnp.bfloat16), w.astype(jnp.bfloat16),
                               (((1,), (0,)), ((), ())),
                               preferred_element_type=jnp.float32)


def _mm_t(x, w):
    return jax.lax.dot_general(x.astype(jnp.bfloat16), w.astype(jnp.bfloat16),
                               (((1,), (1,)), ((), ())),
                               preferred_element_type=jnp.float32)


def _k1_body(x_ref, w1_ref, b1_ref, w2_ref, b2_ref, skw_ref, skb_ref,
             h_ref, sk_ref):
    x = x_ref[...]
    a = jax.nn.relu(_mm(x, w1_ref[...]) + b1_ref[...])
    h = _mm(a, w2_ref[...]) + b2_ref[...]
    blk = x.shape[0]
    ones_col = jnp.where(lax.broadcasted_iota(jnp.int32, (blk, 128), 1) == 0,
                         1.0, 0.0)
    h_ref[...] = jnp.concatenate([h, ones_col], axis=1).astype(jnp.bfloat16)
    sk_ref[...] = (_mm(x, skw_ref[...]) + skb_ref[...]).astype(jnp.bfloat16)


def _run_k1(synthon_feats, rg0_W1, rg0_b1, rg0_W2, rg0_b2, skey_W, skey_b):
    BLK = 512
    grid = (S // BLK,)
    full = lambda shape: pl.BlockSpec(shape, lambda i: (0,) * len(shape))
    return pl.pallas_call(
        _k1_body,
        grid=grid,
        in_specs=[
            pl.BlockSpec((BLK, Q), lambda i: (i, 0)),
            full((Q, H)), full((H,)), full((H, H)), full((H,)),
            full((Q, SK)), full((SK,)),
        ],
        out_specs=[
            pl.BlockSpec((BLK, HA), lambda i: (i, 0)),
            pl.BlockSpec((BLK, SK), lambda i: (i, 0)),
        ],
        out_shape=[
            jax.ShapeDtypeStruct((S, HA), jnp.bfloat16),
            jax.ShapeDtypeStruct((S, SK), jnp.bfloat16),
        ],
    )(synthon_feats, rg0_W1, rg0_b1, rg0_W2, rg0_b2, skey_W, skey_b)


def _make_segsum(n_in, n_out):
    BI = 512
    BO = 512

    def body(idx_ref, x_ref, out_ref):
        j = pl.program_id(0)
        i = pl.program_id(1)

        @pl.when(i == 0)
        def _():
            out_ref[...] = jnp.zeros_like(out_ref)

        idx = idx_ref[0, 0, :]
        local = idx - j * BO
        seg_iota = lax.broadcasted_iota(jnp.int32, (BO, BI), 0)
        oh_t = (seg_iota == local[None, :]).astype(jnp.bfloat16)
        x = x_ref[...]
        out_ref[...] += jax.lax.dot_general(
            oh_t, x, (((1,), (0,)), ((), ())),
            preferred_element_type=jnp.float32)

    def run(data, idx):
        w = data.shape[1]
        idx3 = idx.astype(jnp.int32).reshape(n_in // BI, 1, BI)
        grid = (n_out // BO, n_in // BI)
        return pl.pallas_call(
            body,
            grid=grid,
            in_specs=[
                pl.BlockSpec((1, 1, BI), lambda j, i: (i, 0, 0)),
                pl.BlockSpec((BI, w), lambda j, i: (i, 0)),
            ],
            out_specs=pl.BlockSpec((BO, w), lambda j, i: (j, 0)),
            out_shape=jax.ShapeDtypeStruct((n_out, w), jnp.float32),
        )(idx3, data)

    return run


def _k2_body(pw_ref, w1_ref, b1_ref, w2_ref, b2_ref,
             x1_ref, c1_ref, x2_ref, c2_ref, g_ref):
    pw = pw_ref[...]
    cnt = jnp.maximum(pw[:, H], 1.0)
    pooled = pw[:, :H] / cnt[:, None]
    a = jax.nn.relu(_mm(pooled, w1_ref[...]) + b1_ref[...])
    rf = _mm(a, w2_ref[...]) + b2_ref[...]
    b = jax.nn.relu(_mm(rf, x1_ref[...]) + c1_ref[...])
    g_ref[...] = (_mm(b, x2_ref[...]) + c2_ref[...]).astype(jnp.bfloat16)


def _run_k2(pooled_wide, rg1_W1, rg1_b1, rg1_W2, rg1_b2,
            rx0_W1, rx0_b1, rx0_W2, rx0_b2):
    BLK = 512
    grid = (NR // BLK,)
    full = lambda shape: pl.BlockSpec(shape, lambda i: (0,) * len(shape))
    return pl.pallas_call(
        _k2_body,
        grid=grid,
        in_specs=[
            pl.BlockSpec((BLK, HA), lambda i: (i, 0)),
            full((H, H)), full((H,)), full((H, Q)), full((Q,)),
            full((Q, H)), full((H,)), full((H, H)), full((H,)),
        ],
        out_specs=pl.BlockSpec((BLK, H), lambda i: (i, 0)),
        out_shape=jax.ShapeDtypeStruct((NR, H), jnp.bfloat16),
    )(pooled_wide, rg1_W1, rg1_b1, rg1_W2, rg1_b2,
      rx0_W1, rx0_b1, rx0_W2, rx0_b2)


def _k3_body(rp_ref, w1_ref, b1_ref, w2_ref, b2_ref, kw_ref, kb_ref, out_ref):
    a = jax.nn.relu(_mm(rp_ref[...], w1_ref[...]) + b1_ref[...])
    rf = _mm(a, w2_ref[...]) + b2_ref[...]
    out_ref[...] = (_mm(rf, kw_ref[...]) + kb_ref[...]).astype(jnp.bfloat16)


def _run_k3(reaction_pooled, rx1_W1, rx1_b1, rx1_W2, rx1_b2, rkey_W, rkey_b):
    return pl.pallas_call(
        _k3_body,
        out_shape=jax.ShapeDtypeStruct((R, RK), jnp.bfloat16),
    )(reaction_pooled, rx1_W1, rx1_b1, rx1_W2, rx1_b2, rkey_W, rkey_b)


def _k4_body(x_ref, pw1_ref, pb1_ref, pw2_ref, pb2_ref,
             rw1_ref, rb1_ref, rw2_ref, rb2_ref,
             sw1_ref, sb1_ref, sw2_ref, sb2_ref, rq_ref, sq_ref):
    x = x_ref[...]
    a = jax.nn.relu(_mm(x, pw1_ref[...]) + pb1_ref[...])
    q = x + _mm(a, pw2_ref[...]) + pb2_ref[...]
    ar = jax.nn.relu(_mm(q, rw1_ref[...]) + rb1_ref[...])
    rq_ref[...] = (_mm(ar, rw2_ref[...]) + rb2_ref[...]).astype(jnp.bfloat16)
    asq = jax.nn.relu(_mm(q, sw1_ref[...]) + sb1_ref[...])
    sq_ref[...] = (_mm(asq, sw2_ref[...]) + sb2_ref[...]).astype(jnp.bfloat16)


def _run_k4(queries, proc_W1, proc_b1, proc_W2, proc_b2,
            rq_W1, rq_b1, rq_W2, rq_b2, sq_W1, sq_b1, sq_W2, sq_b2):
    BLK = 512
    grid = (B // BLK,)
    full = lambda shape: pl.BlockSpec(shape, lambda i: (0,) * len(shape))
    return pl.pallas_call(
        _k4_body,
        grid=grid,
        in_specs=[
            pl.BlockSpec((BLK, Q), lambda i: (i, 0)),
            full((Q, H)), full((H,)), full((H, Q)), full((Q,)),
            full((Q, H)), full((H,)), full((H, RK)), full((RK,)),
            full((Q, H)), full((H,)), full((H, SK)), full((SK,)),
        ],
        out_specs=[
            pl.BlockSpec((BLK, RK), lambda i: (i, 0)),
            pl.BlockSpec((BLK, SK), lambda i: (i, 0)),
        ],
        out_shape=[
            jax.ShapeDtypeStruct((B, RK), jnp.bfloat16),
            jax.ShapeDtypeStruct((B, SK), jnp.bfloat16),
        ],
    )(queries, proc_W1, proc_b1, proc_W2, proc_b2,
      rq_W1, rq_b1, rq_W2, rq_b2, sq_W1, sq_b1, sq_W2, sq_b2)


def _log_softmax(x):
    m = jnp.max(x, axis=1, keepdims=True)
    e = jnp.exp(x - m)
    lse = jnp.log(jnp.sum(e, axis=1, keepdims=True)) + m
    return x - lse


def _k5_body(rqa_ref, sqa_ref, rk_ref, sk_ref, out_ref):
    rl = _mm_t(rqa_ref[...], rk_ref[...])
    sl = _mm_t(sqa_ref[...], sk_ref[...])
    out_ref[:, :R] = _log_softmax(rl)
    out_ref[:, R:] = _log_softmax(sl)


def _run_k5(rq_act, sq_act, reaction_keys, synthon_keys):
    BLK = 256
    grid = (B // BLK,)
    full = lambda shape: pl.BlockSpec(shape, lambda i: (0,) * len(shape))
    return pl.pallas_call(
        _k5_body,
        grid=grid,
        in_specs=[
            pl.BlockSpec((BLK, RK), lambda i: (i, 0)),
            pl.BlockSpec((BLK, SK), lambda i: (i, 0)),
            full((R, RK)), full((S, SK)),
        ],
        out_specs=pl.BlockSpec((BLK, R + S), lambda i: (i, 0)),
        out_shape=jax.ShapeDtypeStruct((B, R + S), jnp.float32),
    )(rq_act, sq_act, reaction_keys, synthon_keys)


def kernel(queries, synthon_feats, synthon2rgroup, rgroup2reaction,
           proc_W1, proc_b1, proc_W2, proc_b2,
           rg0_W1, rg0_b1, rg0_W2, rg0_b2,
           rg1_W1, rg1_b1, rg1_W2, rg1_b2,
           rx0_W1, rx0_b1, rx0_W2, rx0_b2,
           rx1_W1, rx1_b1, rx1_W2, rx1_b2,
           rkey_W, rkey_b, skey_W, skey_b,
           rq_W1, rq_b1, rq_W2, rq_b2,
           sq_W1, sq_b1, sq_W2, sq_b2):
    bf = lambda w: w.astype(jnp.bfloat16)
    h_aug, synthon_keys = _run_k1(bf(synthon_feats), bf(rg0_W1), rg0_b1,
                                  bf(rg0_W2), rg0_b2, bf(skey_W), skey_b)
    rq_act, sq_act = _run_k4(queries, bf(proc_W1), proc_b1, bf(proc_W2),
                             proc_b2, bf(rq_W1), rq_b1, bf(rq_W2), rq_b2,
                             bf(sq_W1), sq_b1, bf(sq_W2), sq_b2)

    pooled_wide = _make_segsum(S, NR)(h_aug, synthon2rgroup)
    g = _run_k2(pooled_wide, bf(rg1_W1), rg1_b1, bf(rg1_W2), rg1_b2,
                bf(rx0_W1), rx0_b1, bf(rx0_W2), rx0_b2)
    reaction_pooled = _make_segsum(NR, R)(g, rgroup2reaction)
    reaction_keys = _run_k3(reaction_pooled, bf(rx1_W1), rx1_b1, bf(rx1_W2),
                            rx1_b2, bf(rkey_W), rkey_b)
    return _run_k5(rq_act, sq_act, reaction_keys, synthon_keys)


# speedup vs baseline: 1.2872x; 1.2872x over previous
import functools

import jax
import jax.numpy as jnp
from jax import lax
from jax.experimental import pallas as pl
from jax.experimental.pallas import tpu as pltpu

B, S, NR, R = 2048, 8192, 2048, 512
Q, H, RK, SK = 512, 1024, 128, 128
HA = H + 128


def _mm(x, w):
    return jax.lax.dot_general(x.astype(jnp.bfloat16), w.astype(jnp.bfloat16),
                               (((1,), (0,)), ((), ())),
                               preferred_element_type=jnp.float32)


def _mm_t(x, w):
    return jax.lax.dot_general(x.astype(jnp.bfloat16), w.astype(jnp.bfloat16),
                               (((1,), (1,)), ((), ())),
                               preferred_element_type=jnp.float32)


def _k1_body(x_ref, w1_ref, b1_ref, w2_ref, b2_ref, skw_ref, skb_ref,
             h_ref, sk_ref):
    x = x_ref[...]
    a = jax.nn.relu(_mm(x, w1_ref[...]) + b1_ref[...])
    h = _mm(a, w2_ref[...]) + b2_ref[...]
    blk = x.shape[0]
    ones_col = jnp.where(lax.broadcasted_iota(jnp.int32, (blk, 128), 1) == 0,
                         1.0, 0.0)
    h_ref[...] = jnp.concatenate([h, ones_col], axis=1).astype(jnp.bfloat16)
    sk_ref[...] = (_mm(x, skw_ref[...]) + skb_ref[...]).astype(jnp.bfloat16)


def _run_k1(synthon_feats, rg0_W1, rg0_b1, rg0_W2, rg0_b2, skey_W, skey_b):
    BLK = 512
    grid = (S // BLK,)
    full = lambda shape: pl.BlockSpec(shape, lambda i: (0,) * len(shape))
    return pl.pallas_call(
        _k1_body,
        grid=grid,
        in_specs=[
            pl.BlockSpec((BLK, Q), lambda i: (i, 0)),
            full((Q, H)), full((H,)), full((H, H)), full((H,)),
            full((Q, SK)), full((SK,)),
        ],
        out_specs=[
            pl.BlockSpec((BLK, HA), lambda i: (i, 0)),
            pl.BlockSpec((BLK, SK), lambda i: (i, 0)),
        ],
        out_shape=[
            jax.ShapeDtypeStruct((S, HA), jnp.bfloat16),
            jax.ShapeDtypeStruct((S, SK), jnp.bfloat16),
        ],
    )(synthon_feats, rg0_W1, rg0_b1, rg0_W2, rg0_b2, skey_W, skey_b)


def _make_segsum(n_in, n_out):
    BI = 512
    BO = 512
    NBI = n_in // BI

    def body(start_ref, len_ref, idx_ref, x_ref, out_ref):
        j = pl.program_id(0)
        t = pl.program_id(1)

        @pl.when(t == 0)
        def _():
            out_ref[...] = jnp.zeros_like(out_ref)

        @pl.when(t < len_ref[j])
        def _():
            idx = idx_ref[0, 0, :]
            local = idx - j * BO
            seg_iota = lax.broadcasted_iota(jnp.int32, (BO, BI), 0)
            oh_t = (seg_iota == local[None, :]).astype(jnp.bfloat16)
            x = x_ref[...]
            out_ref[...] += jax.lax.dot_general(
                oh_t, x, (((1,), (0,)), ((), ())),
                preferred_element_type=jnp.float32)

    def run(data, idx):
        w = data.shape[1]
        idx = idx.astype(jnp.int32)
        idx3 = idx.reshape(NBI, 1, BI)
        bounds = jnp.searchsorted(idx, jnp.arange(0, n_out + 1, BO,
                                                  dtype=jnp.int32))
        sb = jnp.minimum(bounds[:-1], n_in - 1) // BI
        eb = jnp.clip(bounds[1:] - 1, 0, n_in - 1) // BI
        eb = jnp.maximum(eb, sb)
        blk_start = sb.astype(jnp.int32)
        blk_len = (eb - sb + 1).astype(jnp.int32)

        def pin(j, t, start, length):
            return jnp.minimum(start[j] + t, start[j] + length[j] - 1)

        grid = (n_out // BO, NBI)
        return pl.pallas_call(
            body,
            grid_spec=pltpu.PrefetchScalarGridSpec(
                num_scalar_prefetch=2,
                grid=grid,
                in_specs=[
                    pl.BlockSpec((1, 1, BI),
                                 lambda j, t, s, l: (pin(j, t, s, l), 0, 0)),
                    pl.BlockSpec((BI, w),
                                 lambda j, t, s, l: (pin(j, t, s, l), 0)),
                ],
                out_specs=pl.BlockSpec((BO, w), lambda j, t, s, l: (j, 0)),
            ),
            out_shape=jax.ShapeDtypeStruct((n_out, w), jnp.float32),
        )(blk_start, blk_len, idx3, data)

    return run


def _k2_body(pw_ref, w1_ref, b1_ref, w2_ref, b2_ref,
             x1_ref, c1_ref, x2_ref, c2_ref, g_ref):
    pw = pw_ref[...]
    cnt = jnp.maximum(pw[:, H], 1.0)
    pooled = pw[:, :H] / cnt[:, None]
    a = jax.nn.relu(_mm(pooled, w1_ref[...]) + b1_ref[...])
    rf = _mm(a, w2_ref[...]) + b2_ref[...]
    b = jax.nn.relu(_mm(rf, x1_ref[...]) + c1_ref[...])
    g_ref[...] = (_mm(b, x2_ref[...]) + c2_ref[...]).astype(jnp.bfloat16)


def _run_k2(pooled_wide, rg1_W1, rg1_b1, rg1_W2, rg1_b2,
            rx0_W1, rx0_b1, rx0_W2, rx0_b2):
    BLK = 512
    grid = (NR // BLK,)
    full = lambda shape: pl.BlockSpec(shape, lambda i: (0,) * len(shape))
    return pl.pallas_call(
        _k2_body,
        grid=grid,
        in_specs=[
            pl.BlockSpec((BLK, HA), lambda i: (i, 0)),
            full((H, H)), full((H,)), full((H, Q)), full((Q,)),
            full((Q, H)), full((H,)), full((H, H)), full((H,)),
        ],
        out_specs=pl.BlockSpec((BLK, H), lambda i: (i, 0)),
        out_shape=jax.ShapeDtypeStruct((NR, H), jnp.bfloat16),
    )(pooled_wide, rg1_W1, rg1_b1, rg1_W2, rg1_b2,
      rx0_W1, rx0_b1, rx0_W2, rx0_b2)


def _k3_body(rp_ref, w1_ref, b1_ref, w2_ref, b2_ref, kw_ref, kb_ref, out_ref):
    a = jax.nn.relu(_mm(rp_ref[...], w1_ref[...]) + b1_ref[...])
    rf = _mm(a, w2_ref[...]) + b2_ref[...]
    out_ref[...] = (_mm(rf, kw_ref[...]) + kb_ref[...]).astype(jnp.bfloat16)


def _run_k3(reaction_pooled, rx1_W1, rx1_b1, rx1_W2, rx1_b2, rkey_W, rkey_b):
    return pl.pallas_call(
        _k3_body,
        out_shape=jax.ShapeDtypeStruct((R, RK), jnp.bfloat16),
    )(reaction_pooled, rx1_W1, rx1_b1, rx1_W2, rx1_b2, rkey_W, rkey_b)


def _k4_body(x_ref, pw1_ref, pb1_ref, pw2_ref, pb2_ref,
             rw1_ref, rb1_ref, rw2_ref, rb2_ref,
             sw1_ref, sb1_ref, sw2_ref, sb2_ref, rq_ref, sq_ref):
    x = x_ref[...]
    a = jax.nn.relu(_mm(x, pw1_ref[...]) + pb1_ref[...])
    q = x + _mm(a, pw2_ref[...]) + pb2_ref[...]
    ar = jax.nn.relu(_mm(q, rw1_ref[...]) + rb1_ref[...])
    rq_ref[...] = (_mm(ar, rw2_ref[...]) + rb2_ref[...]).astype(jnp.bfloat16)
    asq = jax.nn.relu(_mm(q, sw1_ref[...]) + sb1_ref[...])
    sq_ref[...] = (_mm(asq, sw2_ref[...]) + sb2_ref[...]).astype(jnp.bfloat16)


def _run_k4(queries, proc_W1, proc_b1, proc_W2, proc_b2,
            rq_W1, rq_b1, rq_W2, rq_b2, sq_W1, sq_b1, sq_W2, sq_b2):
    BLK = 512
    grid = (B // BLK,)
    full = lambda shape: pl.BlockSpec(shape, lambda i: (0,) * len(shape))
    return pl.pallas_call(
        _k4_body,
        grid=grid,
        in_specs=[
            pl.BlockSpec((BLK, Q), lambda i: (i, 0)),
            full((Q, H)), full((H,)), full((H, Q)), full((Q,)),
            full((Q, H)), full((H,)), full((H, RK)), full((RK,)),
            full((Q, H)), full((H,)), full((H, SK)), full((SK,)),
        ],
        out_specs=[
            pl.BlockSpec((BLK, RK), lambda i: (i, 0)),
            pl.BlockSpec((BLK, SK), lambda i: (i, 0)),
        ],
        out_shape=[
            jax.ShapeDtypeStruct((B, RK), jnp.bfloat16),
            jax.ShapeDtypeStruct((B, SK), jnp.bfloat16),
        ],
    )(queries, proc_W1, proc_b1, proc_W2, proc_b2,
      rq_W1, rq_b1, rq_W2, rq_b2, sq_W1, sq_b1, sq_W2, sq_b2)


def _log_softmax(x):
    m = jnp.max(x, axis=1, keepdims=True)
    e = jnp.exp(x - m)
    lse = jnp.log(jnp.sum(e, axis=1, keepdims=True)) + m
    return x - lse


def _k5_body(rqa_ref, sqa_ref, rk_ref, sk_ref, out_ref):
    rl = _mm_t(rqa_ref[...], rk_ref[...])
    sl = _mm_t(sqa_ref[...], sk_ref[...])
    out_ref[:, :R] = _log_softmax(rl)
    out_ref[:, R:] = _log_softmax(sl)


def _run_k5(rq_act, sq_act, reaction_keys, synthon_keys):
    BLK = 256
    grid = (B // BLK,)
    full = lambda shape: pl.BlockSpec(shape, lambda i: (0,) * len(shape))
    return pl.pallas_call(
        _k5_body,
        grid=grid,
        in_specs=[
            pl.BlockSpec((BLK, RK), lambda i: (i, 0)),
            pl.BlockSpec((BLK, SK), lambda i: (i, 0)),
            full((R, RK)), full((S, SK)),
        ],
        out_specs=pl.BlockSpec((BLK, R + S), lambda i: (i, 0)),
        out_shape=jax.ShapeDtypeStruct((B, R + S), jnp.float32),
    )(rq_act, sq_act, reaction_keys, synthon_keys)


def kernel(queries, synthon_feats, synthon2rgroup, rgroup2reaction,
           proc_W1, proc_b1, proc_W2, proc_b2,
           rg0_W1, rg0_b1, rg0_W2, rg0_b2,
           rg1_W1, rg1_b1, rg1_W2, rg1_b2,
           rx0_W1, rx0_b1, rx0_W2, rx0_b2,
           rx1_W1, rx1_b1, rx1_W2, rx1_b2,
           rkey_W, rkey_b, skey_W, skey_b,
           rq_W1, rq_b1, rq_W2, rq_b2,
           sq_W1, sq_b1, sq_W2, sq_b2):
    h_aug, synthon_keys = _run_k1(synthon_feats, rg0_W1, rg0_b1,
                                  rg0_W2, rg0_b2, skey_W, skey_b)
    rq_act, sq_act = _run_k4(queries, proc_W1, proc_b1, proc_W2, proc_b2,
                             rq_W1, rq_b1, rq_W2, rq_b2,
                             sq_W1, sq_b1, sq_W2, sq_b2)

    pooled_wide = _make_segsum(S, NR)(h_aug, synthon2rgroup)
    g = _run_k2(pooled_wide, rg1_W1, rg1_b1, rg1_W2, rg1_b2,
                rx0_W1, rx0_b1, rx0_W2, rx0_b2)
    reaction_pooled = _make_segsum(NR, R)(g, rgroup2reaction)
    reaction_keys = _run_k3(reaction_pooled, rx1_W1, rx1_b1, rx1_W2, rx1_b2,
                            rkey_W, rkey_b)
    return _run_k5(rq_act, sq_act, reaction_keys, synthon_keys)


# speedup vs baseline: 1.3629x; 1.0588x over previous
import functools

import jax
import jax.numpy as jnp
from jax import lax
from jax.experimental import pallas as pl
from jax.experimental.pallas import tpu as pltpu

B, S, NR, R = 2048, 8192, 2048, 512
Q, H, RK, SK = 512, 1024, 128, 128
HA = H + 128


def _mm(x, w):
    return jax.lax.dot_general(x.astype(jnp.bfloat16), w.astype(jnp.bfloat16),
                               (((1,), (0,)), ((), ())),
                               preferred_element_type=jnp.float32)


def _mm_t(x, w):
    return jax.lax.dot_general(x.astype(jnp.bfloat16), w.astype(jnp.bfloat16),
                               (((1,), (1,)), ((), ())),
                               preferred_element_type=jnp.float32)


def _k1_body(x_ref, w1_ref, b1_ref, skw_ref, skb_ref, h_ref, sk_ref):
    x = x_ref[...]
    a = jax.nn.relu(_mm(x, w1_ref[...]) + b1_ref[...])
    blk = x.shape[0]
    ones_col = jnp.where(lax.broadcasted_iota(jnp.int32, (blk, 128), 1) == 0,
                         1.0, 0.0)
    h_ref[...] = jnp.concatenate([a, ones_col], axis=1).astype(jnp.bfloat16)
    sk_ref[...] = (_mm(x, skw_ref[...]) + skb_ref[...]).astype(jnp.bfloat16)


def _run_k1(synthon_feats, rg0_W1, rg0_b1, skey_W, skey_b):
    BLK = 512
    grid = (S // BLK,)
    full = lambda shape: pl.BlockSpec(shape, lambda i: (0,) * len(shape))
    return pl.pallas_call(
        _k1_body,
        grid=grid,
        in_specs=[
            pl.BlockSpec((BLK, Q), lambda i: (i, 0)),
            full((Q, H)), full((H,)),
            full((Q, SK)), full((SK,)),
        ],
        out_specs=[
            pl.BlockSpec((BLK, HA), lambda i: (i, 0)),
            pl.BlockSpec((BLK, SK), lambda i: (i, 0)),
        ],
        out_shape=[
            jax.ShapeDtypeStruct((S, HA), jnp.bfloat16),
            jax.ShapeDtypeStruct((S, SK), jnp.bfloat16),
        ],
    )(synthon_feats, rg0_W1, rg0_b1, skey_W, skey_b)


def _make_segsum(n_in, n_out):
    BI = 512
    BO = 512
    NBI = n_in // BI

    def body(start_ref, len_ref, idx_ref, x_ref, out_ref):
        j = pl.program_id(0)
        t = pl.program_id(1)

        @pl.when(t == 0)
        def _():
            out_ref[...] = jnp.zeros_like(out_ref)

        @pl.when(t < len_ref[j])
        def _():
            idx = idx_ref[0, 0, :]
            local = idx - j * BO
            seg_iota = lax.broadcasted_iota(jnp.int32, (BO, BI), 0)
            oh_t = (seg_iota == local[None, :]).astype(jnp.bfloat16)
            x = x_ref[...]
            out_ref[...] += jax.lax.dot_general(
                oh_t, x, (((1,), (0,)), ((), ())),
                preferred_element_type=jnp.float32)

    def run(data, idx):
        w = data.shape[1]
        idx = idx.astype(jnp.int32)
        idx3 = idx.reshape(NBI, 1, BI)
        bounds = jnp.searchsorted(idx, jnp.arange(0, n_out + 1, BO,
                                                  dtype=jnp.int32))
        sb = jnp.minimum(bounds[:-1], n_in - 1) // BI
        eb = jnp.clip(bounds[1:] - 1, 0, n_in - 1) // BI
        eb = jnp.maximum(eb, sb)
        blk_start = sb.astype(jnp.int32)
        blk_len = (eb - sb + 1).astype(jnp.int32)

        def pin(j, t, start, length):
            return jnp.minimum(start[j] + t, start[j] + length[j] - 1)

        grid = (n_out // BO, NBI)
        return pl.pallas_call(
            body,
            grid_spec=pltpu.PrefetchScalarGridSpec(
                num_scalar_prefetch=2,
                grid=grid,
                in_specs=[
                    pl.BlockSpec((1, 1, BI),
                                 lambda j, t, s, l: (pin(j, t, s, l), 0, 0)),
                    pl.BlockSpec((BI, w),
                                 lambda j, t, s, l: (pin(j, t, s, l), 0)),
                ],
                out_specs=pl.BlockSpec((BO, w), lambda j, t, s, l: (j, 0)),
            ),
            out_shape=jax.ShapeDtypeStruct((n_out, w), jnp.float32),
        )(blk_start, blk_len, idx3, data)

    return run


def _k2_body(pw_ref, gw2_ref, gb2_ref, w1_ref, b1_ref, w2_ref, b2_ref,
             x1_ref, c1_ref, g_ref):
    pw = pw_ref[...]
    cnt = jnp.maximum(pw[:, H], 1.0)
    mean_a = pw[:, :H] / cnt[:, None]
    rp = _mm(mean_a, gw2_ref[...]) + gb2_ref[...]
    a = jax.nn.relu(_mm(rp, w1_ref[...]) + b1_ref[...])
    rf = _mm(a, w2_ref[...]) + b2_ref[...]
    b = jax.nn.relu(_mm(rf, x1_ref[...]) + c1_ref[...])
    blk = b.shape[0]
    ones_col = jnp.where(lax.broadcasted_iota(jnp.int32, (blk, 128), 1) == 0,
                         1.0, 0.0)
    g_ref[...] = jnp.concatenate([b, ones_col], axis=1).astype(jnp.bfloat16)


def _run_k2(pooled_wide, rg0_W2, rg0_b2, rg1_W1, rg1_b1, rg1_W2, rg1_b2,
            rx0_W1, rx0_b1):
    BLK = 512
    grid = (NR // BLK,)
    full = lambda shape: pl.BlockSpec(shape, lambda i: (0,) * len(shape))
    return pl.pallas_call(
        _k2_body,
        grid=grid,
        in_specs=[
            pl.BlockSpec((BLK, HA), lambda i: (i, 0)),
            full((H, H)), full((H,)),
            full((H, H)), full((H,)), full((H, Q)), full((Q,)),
            full((Q, H)), full((H,)),
        ],
        out_specs=pl.BlockSpec((BLK, HA), lambda i: (i, 0)),
        out_shape=jax.ShapeDtypeStruct((NR, HA), jnp.bfloat16),
    )(pooled_wide, rg0_W2, rg0_b2, rg1_W1, rg1_b1, rg1_W2, rg1_b2,
      rx0_W1, rx0_b1)


def _k3_body(sw_ref, x2_ref, c2_ref, w1_ref, b1_ref, w2_ref, b2_ref,
             kw_ref, kb_ref, out_ref):
    sw = sw_ref[...]
    cnt2 = sw[:, H]
    rp = _mm(sw[:, :H], x2_ref[...]) + cnt2[:, None] * c2_ref[...]
    a = jax.nn.relu(_mm(rp, w1_ref[...]) + b1_ref[...])
    rf = _mm(a, w2_ref[...]) + b2_ref[...]
    out_ref[...] = (_mm(rf, kw_ref[...]) + kb_ref[...]).astype(jnp.bfloat16)


def _run_k3(seg_wide, rx0_W2, rx0_b2, rx1_W1, rx1_b1, rx1_W2, rx1_b2,
            rkey_W, rkey_b):
    return pl.pallas_call(
        _k3_body,
        out_shape=jax.ShapeDtypeStruct((R, RK), jnp.bfloat16),
    )(seg_wide, rx0_W2, rx0_b2, rx1_W1, rx1_b1, rx1_W2, rx1_b2,
      rkey_W, rkey_b)


def _k4_body(x_ref, pw1_ref, pb1_ref, pw2_ref, pb2_ref,
             rw1_ref, rb1_ref, rw2_ref, rb2_ref,
             sw1_ref, sb1_ref, sw2_ref, sb2_ref, rq_ref, sq_ref):
    x = x_ref[...]
    a = jax.nn.relu(_mm(x, pw1_ref[...]) + pb1_ref[...])
    q = x + _mm(a, pw2_ref[...]) + pb2_ref[...]
    ar = jax.nn.relu(_mm(q, rw1_ref[...]) + rb1_ref[...])
    rq_ref[...] = (_mm(ar, rw2_ref[...]) + rb2_ref[...]).astype(jnp.bfloat16)
    asq = jax.nn.relu(_mm(q, sw1_ref[...]) + sb1_ref[...])
    sq_ref[...] = (_mm(asq, sw2_ref[...]) + sb2_ref[...]).astype(jnp.bfloat16)


def _run_k4(queries, proc_W1, proc_b1, proc_W2, proc_b2,
            rq_W1, rq_b1, rq_W2, rq_b2, sq_W1, sq_b1, sq_W2, sq_b2):
    BLK = 512
    grid = (B // BLK,)
    full = lambda shape: pl.BlockSpec(shape, lambda i: (0,) * len(shape))
    return pl.pallas_call(
        _k4_body,
        grid=grid,
        in_specs=[
            pl.BlockSpec((BLK, Q), lambda i: (i, 0)),
            full((Q, H)), full((H,)), full((H, Q)), full((Q,)),
            full((Q, H)), full((H,)), full((H, RK)), full((RK,)),
            full((Q, H)), full((H,)), full((H, SK)), full((SK,)),
        ],
        out_specs=[
            pl.BlockSpec((BLK, RK), lambda i: (i, 0)),
            pl.BlockSpec((BLK, SK), lambda i: (i, 0)),
        ],
        out_shape=[
            jax.ShapeDtypeStruct((B, RK), jnp.bfloat16),
            jax.ShapeDtypeStruct((B, SK), jnp.bfloat16),
        ],
    )(queries, proc_W1, proc_b1, proc_W2, proc_b2,
      rq_W1, rq_b1, rq_W2, rq_b2, sq_W1, sq_b1, sq_W2, sq_b2)


def _log_softmax(x):
    m = jnp.max(x, axis=1, keepdims=True)
    e = jnp.exp(x - m)
    lse = jnp.log(jnp.sum(e, axis=1, keepdims=True)) + m
    return x - lse


def _k5_body(rqa_ref, sqa_ref, rk_ref, sk_ref, out_ref):
    rl = _mm_t(rqa_ref[...], rk_ref[...])
    sl = _mm_t(sqa_ref[...], sk_ref[...])
    out_ref[:, :R] = _log_softmax(rl)
    out_ref[:, R:] = _log_softmax(sl)


def _run_k5(rq_act, sq_act, reaction_keys, synthon_keys):
    BLK = 256
    grid = (B // BLK,)
    full = lambda shape: pl.BlockSpec(shape, lambda i: (0,) * len(shape))
    return pl.pallas_call(
        _k5_body,
        grid=grid,
        in_specs=[
            pl.BlockSpec((BLK, RK), lambda i: (i, 0)),
            pl.BlockSpec((BLK, SK), lambda i: (i, 0)),
            full((R, RK)), full((S, SK)),
        ],
        out_specs=pl.BlockSpec((BLK, R + S), lambda i: (i, 0)),
        out_shape=jax.ShapeDtypeStruct((B, R + S), jnp.float32),
    )(rq_act, sq_act, reaction_keys, synthon_keys)


def kernel(queries, synthon_feats, synthon2rgroup, rgroup2reaction,
           proc_W1, proc_b1, proc_W2, proc_b2,
           rg0_W1, rg0_b1, rg0_W2, rg0_b2,
           rg1_W1, rg1_b1, rg1_W2, rg1_b2,
           rx0_W1, rx0_b1, rx0_W2, rx0_b2,
           rx1_W1, rx1_b1, rx1_W2, rx1_b2,
           rkey_W, rkey_b, skey_W, skey_b,
           rq_W1, rq_b1, rq_W2, rq_b2,
           sq_W1, sq_b1, sq_W2, sq_b2):
    a_aug, synthon_keys = _run_k1(synthon_feats, rg0_W1, rg0_b1,
                                  skey_W, skey_b)
    rq_act, sq_act = _run_k4(queries, proc_W1, proc_b1, proc_W2, proc_b2,
                             rq_W1, rq_b1, rq_W2, rq_b2,
                             sq_W1, sq_b1, sq_W2, sq_b2)

    pooled_wide = _make_segsum(S, NR)(a_aug, synthon2rgroup)
    b_aug = _run_k2(pooled_wide, rg0_W2, rg0_b2, rg1_W1, rg1_b1,
                    rg1_W2, rg1_b2, rx0_W1, rx0_b1)
    seg_wide = _make_segsum(NR, R)(b_aug, rgroup2reaction)
    reaction_keys = _run_k3(seg_wide, rx0_W2, rx0_b2, rx1_W1, rx1_b1,
                            rx1_W2, rx1_b2, rkey_W, rkey_b)
    return _run_k5(rq_act, sq_act, reaction_keys, synthon_keys)


# speedup vs baseline: 1.5659x; 1.1490x over previous
import functools

import jax
import jax.numpy as jnp
from jax import lax
from jax.experimental import pallas as pl
from jax.experimental.pallas import tpu as pltpu

B, S, NR, R = 2048, 8192, 2048, 512
Q, H, RK, SK = 512, 1024, 128, 128
HA = H + 128


def _mm(x, w):
    return jax.lax.dot_general(x.astype(jnp.bfloat16), w.astype(jnp.bfloat16),
                               (((1,), (0,)), ((), ())),
                               preferred_element_type=jnp.float32)


def _mm_t(x, w):
    return jax.lax.dot_general(x.astype(jnp.bfloat16), w.astype(jnp.bfloat16),
                               (((1,), (1,)), ((), ())),
                               preferred_element_type=jnp.float32)


def _k1_body(x_ref, w1_ref, b1_ref, skw_ref, skb_ref, h_ref, sk_ref):
    x = x_ref[...]
    a = jax.nn.relu(_mm(x, w1_ref[...]) + b1_ref[...])
    blk = x.shape[0]
    ones_col = jnp.where(lax.broadcasted_iota(jnp.int32, (blk, 128), 1) == 0,
                         1.0, 0.0)
    h_ref[...] = jnp.concatenate([a, ones_col], axis=1).astype(jnp.bfloat16)
    sk_ref[...] = (_mm(x, skw_ref[...]) + skb_ref[...]).astype(jnp.bfloat16)


def _run_k1(synthon_feats, rg0_W1, rg0_b1, skey_W, skey_b):
    BLK = 512
    grid = (S // BLK,)
    full = lambda shape: pl.BlockSpec(shape, lambda i: (0,) * len(shape))
    return pl.pallas_call(
        _k1_body,
        grid=grid,
        in_specs=[
            pl.BlockSpec((BLK, Q), lambda i: (i, 0)),
            full((Q, H)), full((H,)),
            full((Q, SK)), full((SK,)),
        ],
        out_specs=[
            pl.BlockSpec((BLK, HA), lambda i: (i, 0)),
            pl.BlockSpec((BLK, SK), lambda i: (i, 0)),
        ],
        out_shape=[
            jax.ShapeDtypeStruct((S, HA), jnp.bfloat16),
            jax.ShapeDtypeStruct((S, SK), jnp.bfloat16),
        ],
    )(synthon_feats, rg0_W1, rg0_b1, skey_W, skey_b)


def _make_segsum(n_in, n_out):
    BI = 512
    BO = 512
    NBI = n_in // BI

    def body(start_ref, len_ref, idx_ref, x_ref, out_ref):
        j = pl.program_id(0)
        t = pl.program_id(1)

        @pl.when(t == 0)
        def _():
            out_ref[...] = jnp.zeros_like(out_ref)

        @pl.when(t < len_ref[j])
        def _():
            idx = idx_ref[0, 0, :]
            local = idx - j * BO
            seg_iota = lax.broadcasted_iota(jnp.int32, (BO, BI), 0)
            oh_t = (seg_iota == local[None, :]).astype(jnp.bfloat16)
            x = x_ref[...]
            out_ref[...] += jax.lax.dot_general(
                oh_t, x, (((1,), (0,)), ((), ())),
                preferred_element_type=jnp.float32)

    def run(data, idx):
        w = data.shape[1]
        idx = idx.astype(jnp.int32)
        idx3 = idx.reshape(NBI, 1, BI)
        bounds = jnp.searchsorted(idx, jnp.arange(0, n_out + 1, BO,
                                                  dtype=jnp.int32))
        sb = jnp.minimum(bounds[:-1], n_in - 1) // BI
        eb = jnp.clip(bounds[1:] - 1, 0, n_in - 1) // BI
        eb = jnp.maximum(eb, sb)
        blk_start = sb.astype(jnp.int32)
        blk_len = (eb - sb + 1).astype(jnp.int32)

        def pin(j, t, start, length):
            return jnp.minimum(start[j] + t, start[j] + length[j] - 1)

        grid = (n_out // BO, NBI)
        return pl.pallas_call(
            body,
            grid_spec=pltpu.PrefetchScalarGridSpec(
                num_scalar_prefetch=2,
                grid=grid,
                in_specs=[
                    pl.BlockSpec((1, 1, BI),
                                 lambda j, t, s, l: (pin(j, t, s, l), 0, 0)),
                    pl.BlockSpec((BI, w),
                                 lambda j, t, s, l: (pin(j, t, s, l), 0)),
                ],
                out_specs=pl.BlockSpec((BO, w), lambda j, t, s, l: (j, 0)),
            ),
            out_shape=jax.ShapeDtypeStruct((n_out, w), jnp.float32),
        )(blk_start, blk_len, idx3, data)

    return run


def _k2_body(pw_ref, gw2_ref, gb2_ref, w1_ref, b1_ref, w2_ref, b2_ref,
             x1_ref, c1_ref, g_ref):
    pw = pw_ref[...]
    cnt = jnp.maximum(pw[:, H], 1.0)
    mean_a = pw[:, :H] / cnt[:, None]
    rp = _mm(mean_a, gw2_ref[...]) + gb2_ref[...]
    a = jax.nn.relu(_mm(rp, w1_ref[...]) + b1_ref[...])
    rf = _mm(a, w2_ref[...]) + b2_ref[...]
    b = jax.nn.relu(_mm(rf, x1_ref[...]) + c1_ref[...])
    blk = b.shape[0]
    ones_col = jnp.where(lax.broadcasted_iota(jnp.int32, (blk, 128), 1) == 0,
                         1.0, 0.0)
    g_ref[...] = jnp.concatenate([b, ones_col], axis=1).astype(jnp.bfloat16)


def _run_k2(pooled_wide, rg0_W2, rg0_b2, rg1_W1, rg1_b1, rg1_W2, rg1_b2,
            rx0_W1, rx0_b1):
    BLK = 512
    grid = (NR // BLK,)
    full = lambda shape: pl.BlockSpec(shape, lambda i: (0,) * len(shape))
    return pl.pallas_call(
        _k2_body,
        grid=grid,
        in_specs=[
            pl.BlockSpec((BLK, HA), lambda i: (i, 0)),
            full((H, H)), full((H,)),
            full((H, H)), full((H,)), full((H, Q)), full((Q,)),
            full((Q, H)), full((H,)),
        ],
        out_specs=pl.BlockSpec((BLK, HA), lambda i: (i, 0)),
        out_shape=jax.ShapeDtypeStruct((NR, HA), jnp.bfloat16),
    )(pooled_wide, rg0_W2, rg0_b2, rg1_W1, rg1_b1, rg1_W2, rg1_b2,
      rx0_W1, rx0_b1)


def _k3_body(idx_ref, b_ref, x2_ref, c2_ref, w1_ref, b1_ref, w2_ref, b2_ref,
             kw_ref, kb_ref, out_ref):
    idx = idx_ref[0, 0, :]
    oh = (lax.broadcasted_iota(jnp.int32, (R, NR), 0)
          == idx[None, :]).astype(jnp.bfloat16)
    sw = jax.lax.dot_general(oh, b_ref[...], (((1,), (0,)), ((), ())),
                             preferred_element_type=jnp.float32)
    cnt2 = sw[:, H]
    rp = _mm(sw[:, :H], x2_ref[...]) + cnt2[:, None] * c2_ref[...]
    a = jax.nn.relu(_mm(rp, w1_ref[...]) + b1_ref[...])
    rf = _mm(a, w2_ref[...]) + b2_ref[...]
    out_ref[...] = (_mm(rf, kw_ref[...]) + kb_ref[...]).astype(jnp.bfloat16)


def _run_k3(b_aug, rgroup2reaction, rx0_W2, rx0_b2, rx1_W1, rx1_b1,
            rx1_W2, rx1_b2, rkey_W, rkey_b):
    idx3 = rgroup2reaction.astype(jnp.int32).reshape(1, 1, NR)
    return pl.pallas_call(
        _k3_body,
        out_shape=jax.ShapeDtypeStruct((R, RK), jnp.bfloat16),
    )(idx3, b_aug, rx0_W2, rx0_b2, rx1_W1, rx1_b1, rx1_W2, rx1_b2,
      rkey_W, rkey_b)


def _k4_body(x_ref, pw1_ref, pb1_ref, pw2_ref, pb2_ref,
             rw1_ref, rb1_ref, rw2_ref, rb2_ref,
             sw1_ref, sb1_ref, sw2_ref, sb2_ref, rq_ref, sq_ref):
    x = x_ref[...]
    a = jax.nn.relu(_mm(x, pw1_ref[...]) + pb1_ref[...])
    q = x + _mm(a, pw2_ref[...]) + pb2_ref[...]
    ar = jax.nn.relu(_mm(q, rw1_ref[...]) + rb1_ref[...])
    rq_ref[...] = (_mm(ar, rw2_ref[...]) + rb2_ref[...]).astype(jnp.bfloat16)
    asq = jax.nn.relu(_mm(q, sw1_ref[...]) + sb1_ref[...])
    sq_ref[...] = (_mm(asq, sw2_ref[...]) + sb2_ref[...]).astype(jnp.bfloat16)


def _run_k4(queries, proc_W1, proc_b1, proc_W2, proc_b2,
            rq_W1, rq_b1, rq_W2, rq_b2, sq_W1, sq_b1, sq_W2, sq_b2):
    BLK = 512
    grid = (B // BLK,)
    full = lambda shape: pl.BlockSpec(shape, lambda i: (0,) * len(shape))
    return pl.pallas_call(
        _k4_body,
        grid=grid,
        in_specs=[
            pl.BlockSpec((BLK, Q), lambda i: (i, 0)),
            full((Q, H)), full((H,)), full((H, Q)), full((Q,)),
            full((Q, H)), full((H,)), full((H, RK)), full((RK,)),
            full((Q, H)), full((H,)), full((H, SK)), full((SK,)),
        ],
        out_specs=[
            pl.BlockSpec((BLK, RK), lambda i: (i, 0)),
            pl.BlockSpec((BLK, SK), lambda i: (i, 0)),
        ],
        out_shape=[
            jax.ShapeDtypeStruct((B, RK), jnp.bfloat16),
            jax.ShapeDtypeStruct((B, SK), jnp.bfloat16),
        ],
    )(queries, proc_W1, proc_b1, proc_W2, proc_b2,
      rq_W1, rq_b1, rq_W2, rq_b2, sq_W1, sq_b1, sq_W2, sq_b2)


def _log_softmax(x):
    lse = jnp.log(jnp.sum(jnp.exp(x), axis=1, keepdims=True))
    return x - lse


def _k5_body(rqa_ref, sqa_ref, rk_ref, sk_ref, out_ref):
    rl = _mm_t(rqa_ref[...], rk_ref[...])
    sl = _mm_t(sqa_ref[...], sk_ref[...])
    out_ref[:, :R] = _log_softmax(rl)
    out_ref[:, R:] = _log_softmax(sl)


def _run_k5(rq_act, sq_act, reaction_keys, synthon_keys):
    BLK = 256
    grid = (B // BLK,)
    full = lambda shape: pl.BlockSpec(shape, lambda i: (0,) * len(shape))
    return pl.pallas_call(
        _k5_body,
        grid=grid,
        in_specs=[
            pl.BlockSpec((BLK, RK), lambda i: (i, 0)),
            pl.BlockSpec((BLK, SK), lambda i: (i, 0)),
            full((R, RK)), full((S, SK)),
        ],
        out_specs=pl.BlockSpec((BLK, R + S), lambda i: (i, 0)),
        out_shape=jax.ShapeDtypeStruct((B, R + S), jnp.float32),
    )(rq_act, sq_act, reaction_keys, synthon_keys)


def kernel(queries, synthon_feats, synthon2rgroup, rgroup2reaction,
           proc_W1, proc_b1, proc_W2, proc_b2,
           rg0_W1, rg0_b1, rg0_W2, rg0_b2,
           rg1_W1, rg1_b1, rg1_W2, rg1_b2,
           rx0_W1, rx0_b1, rx0_W2, rx0_b2,
           rx1_W1, rx1_b1, rx1_W2, rx1_b2,
           rkey_W, rkey_b, skey_W, skey_b,
           rq_W1, rq_b1, rq_W2, rq_b2,
           sq_W1, sq_b1, sq_W2, sq_b2):
    a_aug, synthon_keys = _run_k1(synthon_feats, rg0_W1, rg0_b1,
                                  skey_W, skey_b)
    rq_act, sq_act = _run_k4(queries, proc_W1, proc_b1, proc_W2, proc_b2,
                             rq_W1, rq_b1, rq_W2, rq_b2,
                             sq_W1, sq_b1, sq_W2, sq_b2)

    pooled_wide = _make_segsum(S, NR)(a_aug, synthon2rgroup)
    b_aug = _run_k2(pooled_wide, rg0_W2, rg0_b2, rg1_W1, rg1_b1,
                    rg1_W2, rg1_b2, rx0_W1, rx0_b1)
    reaction_keys = _run_k3(b_aug, rgroup2reaction, rx0_W2, rx0_b2,
                            rx1_W1, rx1_b1, rx1_W2, rx1_b2, rkey_W, rkey_b)
    return _run_k5(rq_act, sq_act, reaction_keys, synthon_keys)


# speedup vs baseline: 1.6544x; 1.0565x over previous
import functools

import jax
import jax.numpy as jnp
from jax import lax
from jax.experimental import pallas as pl
from jax.experimental.pallas import tpu as pltpu

B, S, NR, R = 2048, 8192, 2048, 512
Q, H, RK, SK = 512, 1024, 128, 128
HA = H + 128


def _mm(x, w):
    return jax.lax.dot_general(x.astype(jnp.bfloat16), w.astype(jnp.bfloat16),
                               (((1,), (0,)), ((), ())),
                               preferred_element_type=jnp.float32)


def _mm_t(x, w):
    return jax.lax.dot_general(x.astype(jnp.bfloat16), w.astype(jnp.bfloat16),
                               (((1,), (1,)), ((), ())),
                               preferred_element_type=jnp.float32)


def _k1_body(x_ref, w1_ref, b1_ref, skw_ref, skb_ref, h_ref, sk_ref):
    x = x_ref[...]
    a = jax.nn.relu(_mm(x, w1_ref[...]) + b1_ref[...])
    blk = x.shape[0]
    ones_col = jnp.where(lax.broadcasted_iota(jnp.int32, (blk, 128), 1) == 0,
                         1.0, 0.0)
    h_ref[...] = jnp.concatenate([a, ones_col], axis=1).astype(jnp.bfloat16)
    sk_ref[...] = (_mm(x, skw_ref[...]) + skb_ref[...]).astype(jnp.bfloat16)


def _run_k1(synthon_feats, rg0_W1, rg0_b1, skey_W, skey_b):
    BLK = 512
    grid = (S // BLK,)
    full = lambda shape: pl.BlockSpec(shape, lambda i: (0,) * len(shape))
    return pl.pallas_call(
        _k1_body,
        grid=grid,
        in_specs=[
            pl.BlockSpec((BLK, Q), lambda i: (i, 0)),
            full((Q, H)), full((H,)),
            full((Q, SK)), full((SK,)),
        ],
        out_specs=[
            pl.BlockSpec((BLK, HA), lambda i: (i, 0)),
            pl.BlockSpec((BLK, SK), lambda i: (i, 0)),
        ],
        out_shape=[
            jax.ShapeDtypeStruct((S, HA), jnp.bfloat16),
            jax.ShapeDtypeStruct((S, SK), jnp.bfloat16),
        ],
    )(synthon_feats, rg0_W1, rg0_b1, skey_W, skey_b)


def _make_segsum(n_in, n_out):
    BI = 512
    BO = 512
    NBI = n_in // BI

    def body(start_ref, len_ref, idx_ref, x_ref, out_ref):
        j = pl.program_id(0)
        t = pl.program_id(1)

        @pl.when(t == 0)
        def _():
            out_ref[...] = jnp.zeros_like(out_ref)

        @pl.when(t < len_ref[j])
        def _():
            idx = idx_ref[0, 0, :]
            local = idx - j * BO
            seg_iota = lax.broadcasted_iota(jnp.int32, (BO, BI), 0)
            oh_t = (seg_iota == local[None, :]).astype(jnp.bfloat16)
            x = x_ref[...]
            out_ref[...] += jax.lax.dot_general(
                oh_t, x, (((1,), (0,)), ((), ())),
                preferred_element_type=jnp.float32)

    def run(data, idx):
        w = data.shape[1]
        idx = idx.astype(jnp.int32)
        idx3 = idx.reshape(NBI, 1, BI)
        bounds = jnp.searchsorted(idx, jnp.arange(0, n_out + 1, BO,
                                                  dtype=jnp.int32))
        sb = jnp.minimum(bounds[:-1], n_in - 1) // BI
        eb = jnp.clip(bounds[1:] - 1, 0, n_in - 1) // BI
        eb = jnp.maximum(eb, sb)
        blk_start = sb.astype(jnp.int32)
        blk_len = (eb - sb + 1).astype(jnp.int32)

        def pin(j, t, start, length):
            return jnp.minimum(start[j] + t, start[j] + length[j] - 1)

        grid = (n_out // BO, NBI)
        return pl.pallas_call(
            body,
            grid_spec=pltpu.PrefetchScalarGridSpec(
                num_scalar_prefetch=2,
                grid=grid,
                in_specs=[
                    pl.BlockSpec((1, 1, BI),
                                 lambda j, t, s, l: (pin(j, t, s, l), 0, 0)),
                    pl.BlockSpec((BI, w),
                                 lambda j, t, s, l: (pin(j, t, s, l), 0)),
                ],
                out_specs=pl.BlockSpec((BO, w), lambda j, t, s, l: (j, 0)),
            ),
            out_shape=jax.ShapeDtypeStruct((n_out, w), jnp.float32),
        )(blk_start, blk_len, idx3, data)

    return run


def _k2_body(pw_ref, gw2_ref, gb2_ref, w1_ref, b1_ref, w2_ref, b2_ref,
             x1_ref, c1_ref, g_ref):
    pw = pw_ref[...]
    cnt = jnp.maximum(pw[:, H], 1.0)
    mean_a = pw[:, :H] / cnt[:, None]
    rp = _mm(mean_a, gw2_ref[...]) + gb2_ref[...]
    a = jax.nn.relu(_mm(rp, w1_ref[...]) + b1_ref[...])
    rf = _mm(a, w2_ref[...]) + b2_ref[...]
    b = jax.nn.relu(_mm(rf, x1_ref[...]) + c1_ref[...])
    blk = b.shape[0]
    ones_col = jnp.where(lax.broadcasted_iota(jnp.int32, (blk, 128), 1) == 0,
                         1.0, 0.0)
    g_ref[...] = jnp.concatenate([b, ones_col], axis=1).astype(jnp.bfloat16)


def _run_k2(pooled_wide, rg0_W2, rg0_b2, rg1_W1, rg1_b1, rg1_W2, rg1_b2,
            rx0_W1, rx0_b1):
    BLK = 512
    grid = (NR // BLK,)
    full = lambda shape: pl.BlockSpec(shape, lambda i: (0,) * len(shape))
    return pl.pallas_call(
        _k2_body,
        grid=grid,
        in_specs=[
            pl.BlockSpec((BLK, HA), lambda i: (i, 0)),
            full((H, H)), full((H,)),
            full((H, H)), full((H,)), full((H, Q)), full((Q,)),
            full((Q, H)), full((H,)),
        ],
        out_specs=pl.BlockSpec((BLK, HA), lambda i: (i, 0)),
        out_shape=jax.ShapeDtypeStruct((NR, HA), jnp.bfloat16),
    )(pooled_wide, rg0_W2, rg0_b2, rg1_W1, rg1_b1, rg1_W2, rg1_b2,
      rx0_W1, rx0_b1)


def _k3_body(idx_ref, b_ref, x2_ref, c2_ref, w1_ref, b1_ref, w2_ref, b2_ref,
             kw_ref, kb_ref, out_ref):
    idx = idx_ref[0, 0, :]
    oh = (lax.broadcasted_iota(jnp.int32, (R, NR), 0)
          == idx[None, :]).astype(jnp.bfloat16)
    sw = jax.lax.dot_general(oh, b_ref[...], (((1,), (0,)), ((), ())),
                             preferred_element_type=jnp.float32)
    cnt2 = sw[:, H]
    rp = _mm(sw[:, :H], x2_ref[...]) + cnt2[:, None] * c2_ref[...]
    a = jax.nn.relu(_mm(rp, w1_ref[...]) + b1_ref[...])
    rf = _mm(a, w2_ref[...]) + b2_ref[...]
    out_ref[...] = (_mm(rf, kw_ref[...]) + kb_ref[...]).astype(jnp.bfloat16)


def _run_k3(b_aug, rgroup2reaction, rx0_W2, rx0_b2, rx1_W1, rx1_b1,
            rx1_W2, rx1_b2, rkey_W, rkey_b):
    idx3 = rgroup2reaction.astype(jnp.int32).reshape(1, 1, NR)
    return pl.pallas_call(
        _k3_body,
        out_shape=jax.ShapeDtypeStruct((R, RK), jnp.bfloat16),
    )(idx3, b_aug, rx0_W2, rx0_b2, rx1_W1, rx1_b1, rx1_W2, rx1_b2,
      rkey_W, rkey_b)


def _log_softmax(x):
    lse = jnp.log(jnp.sum(jnp.exp(x), axis=1, keepdims=True))
    return x - lse


def _k5_body(x_ref, pw1_ref, pb1_ref, pw2_ref, pb2_ref,
             rw1_ref, rb1_ref, rw2_ref, rb2_ref,
             sw1_ref, sb1_ref, sw2_ref, sb2_ref,
             rk_ref, sk_ref, out_ref):
    x = x_ref[...]
    a = jax.nn.relu(_mm(x, pw1_ref[...]) + pb1_ref[...])
    q = x + _mm(a, pw2_ref[...]) + pb2_ref[...]
    ar = jax.nn.relu(_mm(q, rw1_ref[...]) + rb1_ref[...])
    rqa = _mm(ar, rw2_ref[...]) + rb2_ref[...]
    asq = jax.nn.relu(_mm(q, sw1_ref[...]) + sb1_ref[...])
    sqa = _mm(asq, sw2_ref[...]) + sb2_ref[...]
    rl = _mm_t(rqa, rk_ref[...])
    sl = _mm_t(sqa, sk_ref[...])
    out_ref[:, :R] = _log_softmax(rl)
    out_ref[:, R:] = _log_softmax(sl)


def _run_k5(queries, proc_W1, proc_b1, proc_W2, proc_b2,
            rq_W1, rq_b1, rq_W2, rq_b2, sq_W1, sq_b1, sq_W2, sq_b2,
            reaction_keys, synthon_keys):
    BLK = 256
    grid = (B // BLK,)
    full = lambda shape: pl.BlockSpec(shape, lambda i: (0,) * len(shape))
    return pl.pallas_call(
        _k5_body,
        grid=grid,
        in_specs=[
            pl.BlockSpec((BLK, Q), lambda i: (i, 0)),
            full((Q, H)), full((H,)), full((H, Q)), full((Q,)),
            full((Q, H)), full((H,)), full((H, RK)), full((RK,)),
            full((Q, H)), full((H,)), full((H, SK)), full((SK,)),
            full((R, RK)), full((S, SK)),
        ],
        out_specs=pl.BlockSpec((BLK, R + S), lambda i: (i, 0)),
        out_shape=jax.ShapeDtypeStruct((B, R + S), jnp.float32),
    )(queries, proc_W1, proc_b1, proc_W2, proc_b2,
      rq_W1, rq_b1, rq_W2, rq_b2, sq_W1, sq_b1, sq_W2, sq_b2,
      reaction_keys, synthon_keys)


def kernel(queries, synthon_feats, synthon2rgroup, rgroup2reaction,
           proc_W1, proc_b1, proc_W2, proc_b2,
           rg0_W1, rg0_b1, rg0_W2, rg0_b2,
           rg1_W1, rg1_b1, rg1_W2, rg1_b2,
           rx0_W1, rx0_b1, rx0_W2, rx0_b2,
           rx1_W1, rx1_b1, rx1_W2, rx1_b2,
           rkey_W, rkey_b, skey_W, skey_b,
           rq_W1, rq_b1, rq_W2, rq_b2,
           sq_W1, sq_b1, sq_W2, sq_b2):
    a_aug, synthon_keys = _run_k1(synthon_feats, rg0_W1, rg0_b1,
                                  skey_W, skey_b)
    pooled_wide = _make_segsum(S, NR)(a_aug, synthon2rgroup)
    b_aug = _run_k2(pooled_wide, rg0_W2, rg0_b2, rg1_W1, rg1_b1,
                    rg1_W2, rg1_b2, rx0_W1, rx0_b1)
    reaction_keys = _run_k3(b_aug, rgroup2reaction, rx0_W2, rx0_b2,
                            rx1_W1, rx1_b1, rx1_W2, rx1_b2, rkey_W, rkey_b)
    return _run_k5(queries, proc_W1, proc_b1, proc_W2, proc_b2,
                   rq_W1, rq_b1, rq_W2, rq_b2, sq_W1, sq_b1, sq_W2, sq_b2,
                   reaction_keys, synthon_keys)


# speedup vs baseline: 1.7811x; 1.0766x over previous
import functools

import jax
import jax.numpy as jnp
from jax import lax
from jax.experimental import pallas as pl
from jax.experimental.pallas import tpu as pltpu

B, S, NR, R = 2048, 8192, 2048, 512
Q, H, RK, SK = 512, 1024, 128, 128
HA = H + 128


def _mm(x, w):
    return jax.lax.dot_general(x.astype(jnp.bfloat16), w.astype(jnp.bfloat16),
                               (((1,), (0,)), ((), ())),
                               preferred_element_type=jnp.float32)


def _mm_t(x, w):
    return jax.lax.dot_general(x.astype(jnp.bfloat16), w.astype(jnp.bfloat16),
                               (((1,), (1,)), ((), ())),
                               preferred_element_type=jnp.float32)


def _k1_body(x_ref, skw_ref, skb_ref, sk_ref):
    sk_ref[...] = (_mm(x_ref[...], skw_ref[...]) + skb_ref[...]).astype(jnp.bfloat16)


def _run_k1(synthon_feats, skey_W, skey_b):
    BLK = 2048
    grid = (S // BLK,)
    full = lambda shape: pl.BlockSpec(shape, lambda i: (0,) * len(shape))
    return pl.pallas_call(
        _k1_body,
        grid=grid,
        in_specs=[
            pl.BlockSpec((BLK, Q), lambda i: (i, 0)),
            full((Q, SK)), full((SK,)),
        ],
        out_specs=pl.BlockSpec((BLK, SK), lambda i: (i, 0)),
        out_shape=jax.ShapeDtypeStruct((S, SK), jnp.bfloat16),
    )(synthon_feats, skey_W, skey_b)


def _make_fused_segsum():
    BI = 512
    BO = 512
    NBI = S // BI

    def body(start_ref, len_ref, idx_ref, x_ref, w1_ref, b1_ref, out_ref):
        j = pl.program_id(0)
        t = pl.program_id(1)

        @pl.when(t == 0)
        def _():
            out_ref[...] = jnp.zeros_like(out_ref)

        @pl.when(t < len_ref[j])
        def _():
            idx = idx_ref[0, 0, :]
            local = idx - j * BO
            seg_iota = lax.broadcasted_iota(jnp.int32, (BO, BI), 0)
            oh_t = (seg_iota == local[None, :]).astype(jnp.bfloat16)
            a = jax.nn.relu(_mm(x_ref[...], w1_ref[...]) + b1_ref[...])
            ones_col = jnp.where(
                lax.broadcasted_iota(jnp.int32, (BI, 128), 1) == 0, 1.0, 0.0)
            x = jnp.concatenate([a.astype(jnp.bfloat16),
                                 ones_col.astype(jnp.bfloat16)], axis=1)
            out_ref[...] += jax.lax.dot_general(
                oh_t, x, (((1,), (0,)), ((), ())),
                preferred_element_type=jnp.float32)

    def run(synthon_feats, rg0_W1, rg0_b1, idx):
        idx = idx.astype(jnp.int32)
        idx3 = idx.reshape(NBI, 1, BI)
        bounds = jnp.searchsorted(idx, jnp.arange(0, NR + 1, BO,
                                                  dtype=jnp.int32))
        sb = jnp.minimum(bounds[:-1], S - 1) // BI
        eb = jnp.clip(bounds[1:] - 1, 0, S - 1) // BI
        eb = jnp.maximum(eb, sb)
        blk_start = sb.astype(jnp.int32)
        blk_len = (eb - sb + 1).astype(jnp.int32)

        def pin(j, t, start, length):
            return jnp.minimum(start[j] + t, start[j] + length[j] - 1)

        grid = (NR // BO, NBI)
        full = lambda shape: pl.BlockSpec(shape, lambda *a: (0,) * len(shape))
        return pl.pallas_call(
            body,
            grid_spec=pltpu.PrefetchScalarGridSpec(
                num_scalar_prefetch=2,
                grid=grid,
                in_specs=[
                    pl.BlockSpec((1, 1, BI),
                                 lambda j, t, s, l: (pin(j, t, s, l), 0, 0)),
                    pl.BlockSpec((BI, Q),
                                 lambda j, t, s, l: (pin(j, t, s, l), 0)),
                    full((Q, H)), full((H,)),
                ],
                out_specs=pl.BlockSpec((BO, HA), lambda j, t, s, l: (j, 0)),
            ),
            out_shape=jax.ShapeDtypeStruct((NR, HA), jnp.float32),
        )(blk_start, blk_len, idx3, synthon_feats, rg0_W1, rg0_b1)

    return run


def _k23_body(idx_ref, pw_ref, gw2_ref, gb2_ref, w1_ref, b1_ref,
              w2_ref, b2_ref, x1_ref, c1_ref,
              x2_ref, c2_ref, rw1_ref, rb1_ref, rw2_ref, rb2_ref,
              kw_ref, kb_ref, out_ref, b_scr):
    i = pl.program_id(0)
    pw = pw_ref[...]
    cnt = jnp.maximum(pw[:, H], 1.0)
    mean_a = pw[:, :H] / cnt[:, None]
    rp = _mm(mean_a, gw2_ref[...]) + gb2_ref[...]
    a = jax.nn.relu(_mm(rp, w1_ref[...]) + b1_ref[...])
    rf = _mm(a, w2_ref[...]) + b2_ref[...]
    b = jax.nn.relu(_mm(rf, x1_ref[...]) + c1_ref[...])
    blk = b.shape[0]
    ones_col = jnp.where(lax.broadcasted_iota(jnp.int32, (blk, 128), 1) == 0,
                         1.0, 0.0)
    b_scr[pl.ds(i * blk, blk), :] = jnp.concatenate(
        [b, ones_col], axis=1).astype(jnp.bfloat16)

    @pl.when(i == NR // blk - 1)
    def _():
        idx = idx_ref[0, 0, :]
        oh = (lax.broadcasted_iota(jnp.int32, (R, NR), 0)
              == idx[None, :]).astype(jnp.bfloat16)
        sw = jax.lax.dot_general(oh, b_scr[...], (((1,), (0,)), ((), ())),
                                 preferred_element_type=jnp.float32)
        cnt2 = sw[:, H]
        rp2 = _mm(sw[:, :H], x2_ref[...]) + cnt2[:, None] * c2_ref[...]
        a2 = jax.nn.relu(_mm(rp2, rw1_ref[...]) + rb1_ref[...])
        rf2 = _mm(a2, rw2_ref[...]) + rb2_ref[...]
        out_ref[...] = (_mm(rf2, kw_ref[...]) + kb_ref[...]).astype(jnp.bfloat16)


def _run_k23(pooled_wide, rgroup2reaction, rg0_W2, rg0_b2,
             rg1_W1, rg1_b1, rg1_W2, rg1_b2, rx0_W1, rx0_b1,
             rx0_W2, rx0_b2, rx1_W1, rx1_b1, rx1_W2, rx1_b2,
             rkey_W, rkey_b):
    BLK = 512
    grid = (NR // BLK,)
    full = lambda shape: pl.BlockSpec(shape, lambda i: (0,) * len(shape))
    idx3 = rgroup2reaction.astype(jnp.int32).reshape(1, 1, NR)
    return pl.pallas_call(
        _k23_body,
        grid=grid,
        in_specs=[
            full((1, 1, NR)),
            pl.BlockSpec((BLK, HA), lambda i: (i, 0)),
            full((H, H)), full((H,)),
            full((H, H)), full((H,)), full((H, Q)), full((Q,)),
            full((Q, H)), full((H,)),
            full((H, H)), full((H,)), full((H, H)), full((H,)),
            full((H, Q)), full((Q,)),
            full((Q, RK)), full((RK,)),
        ],
        out_specs=pl.BlockSpec((R, RK), lambda i: (0, 0)),
        out_shape=jax.ShapeDtypeStruct((R, RK), jnp.bfloat16),
        scratch_shapes=[pltpu.VMEM((NR, HA), jnp.bfloat16)],
    )(idx3, pooled_wide, rg0_W2, rg0_b2, rg1_W1, rg1_b1, rg1_W2, rg1_b2,
      rx0_W1, rx0_b1, rx0_W2, rx0_b2, rx1_W1, rx1_b1, rx1_W2, rx1_b2,
      rkey_W, rkey_b)


def _log_softmax(x):
    lse = jnp.log(jnp.sum(jnp.exp(x), axis=1, keepdims=True))
    return x - lse


def _k5_body(x_ref, pw1_ref, pb1_ref, pw2_ref, pb2_ref,
             rw1_ref, rb1_ref, rw2_ref, rb2_ref,
             sw1_ref, sb1_ref, sw2_ref, sb2_ref,
             rk_ref, sk_ref, out_ref):
    x = x_ref[...]
    a = jax.nn.relu(_mm(x, pw1_ref[...]) + pb1_ref[...])
    q = x + _mm(a, pw2_ref[...]) + pb2_ref[...]
    ar = jax.nn.relu(_mm(q, rw1_ref[...]) + rb1_ref[...])
    rqa = _mm(ar, rw2_ref[...]) + rb2_ref[...]
    asq = jax.nn.relu(_mm(q, sw1_ref[...]) + sb1_ref[...])
    sqa = _mm(asq, sw2_ref[...]) + sb2_ref[...]
    rl = _mm_t(rqa, rk_ref[...])
    sl = _mm_t(sqa, sk_ref[...])
    out_ref[:, :R] = _log_softmax(rl)
    out_ref[:, R:] = _log_softmax(sl)


def _run_k5(queries, proc_W1, proc_b1, proc_W2, proc_b2,
            rq_W1, rq_b1, rq_W2, rq_b2, sq_W1, sq_b1, sq_W2, sq_b2,
            reaction_keys, synthon_keys):
    BLK = 256
    grid = (B // BLK,)
    full = lambda shape: pl.BlockSpec(shape, lambda i: (0,) * len(shape))
    return pl.pallas_call(
        _k5_body,
        grid=grid,
        in_specs=[
            pl.BlockSpec((BLK, Q), lambda i: (i, 0)),
            full((Q, H)), full((H,)), full((H, Q)), full((Q,)),
            full((Q, H)), full((H,)), full((H, RK)), full((RK,)),
            full((Q, H)), full((H,)), full((H, SK)), full((SK,)),
            full((R, RK)), full((S, SK)),
        ],
        out_specs=pl.BlockSpec((BLK, R + S), lambda i: (i, 0)),
        out_shape=jax.ShapeDtypeStruct((B, R + S), jnp.float32),
    )(queries, proc_W1, proc_b1, proc_W2, proc_b2,
      rq_W1, rq_b1, rq_W2, rq_b2, sq_W1, sq_b1, sq_W2, sq_b2,
      reaction_keys, synthon_keys)


def kernel(queries, synthon_feats, synthon2rgroup, rgroup2reaction,
           proc_W1, proc_b1, proc_W2, proc_b2,
           rg0_W1, rg0_b1, rg0_W2, rg0_b2,
           rg1_W1, rg1_b1, rg1_W2, rg1_b2,
           rx0_W1, rx0_b1, rx0_W2, rx0_b2,
           rx1_W1, rx1_b1, rx1_W2, rx1_b2,
           rkey_W, rkey_b, skey_W, skey_b,
           rq_W1, rq_b1, rq_W2, rq_b2,
           sq_W1, sq_b1, sq_W2, sq_b2):
    synthon_keys = _run_k1(synthon_feats, skey_W, skey_b)
    pooled_wide = _make_fused_segsum()(synthon_feats, rg0_W1, rg0_b1,
                                       synthon2rgroup)
    reaction_keys = _run_k23(pooled_wide, rgroup2reaction, rg0_W2, rg0_b2,
                             rg1_W1, rg1_b1, rg1_W2, rg1_b2, rx0_W1, rx0_b1,
                             rx0_W2, rx0_b2, rx1_W1, rx1_b1, rx1_W2, rx1_b2,
                             rkey_W, rkey_b)
    return _run_k5(queries, proc_W1, proc_b1, proc_W2, proc_b2,
                   rq_W1, rq_b1, rq_W2, rq_b2, sq_W1, sq_b1, sq_W2, sq_b2,
                   reaction_keys, synthon_keys)


# speedup vs baseline: 1.8282x; 1.0264x over previous
import functools

import jax
import jax.numpy as jnp
from jax import lax
from jax.experimental import pallas as pl
from jax.experimental.pallas import tpu as pltpu

B, S, NR, R = 2048, 8192, 2048, 512
Q, H, RK, SK = 512, 1024, 128, 128
HA = H + 128


def _mm(x, w):
    return jax.lax.dot_general(x.astype(jnp.bfloat16), w.astype(jnp.bfloat16),
                               (((1,), (0,)), ((), ())),
                               preferred_element_type=jnp.float32)


def _mm_t(x, w):
    return jax.lax.dot_general(x.astype(jnp.bfloat16), w.astype(jnp.bfloat16),
                               (((1,), (1,)), ((), ())),
                               preferred_element_type=jnp.float32)


def _make_fused_segsum():
    BI = 512
    BO = 512
    NBI = S // BI

    def body(start_ref, len_ref, idx_ref, x_ref, w1_ref, b1_ref,
             skw_ref, skb_ref, out_ref, sk_ref):
        j = pl.program_id(0)
        t = pl.program_id(1)

        @pl.when(t == 0)
        def _():
            out_ref[...] = jnp.zeros_like(out_ref)

        @pl.when(t < len_ref[j])
        def _():
            idx = idx_ref[0, 0, :]
            local = idx - j * BO
            seg_iota = lax.broadcasted_iota(jnp.int32, (BO, BI), 0)
            oh_t = (seg_iota == local[None, :]).astype(jnp.bfloat16)
            xv = x_ref[...]
            sk_ref[...] = (_mm(xv, skw_ref[...]) + skb_ref[...]).astype(jnp.bfloat16)
            a = jax.nn.relu(_mm(xv, w1_ref[...]) + b1_ref[...])
            ones_col = jnp.where(
                lax.broadcasted_iota(jnp.int32, (BI, 128), 1) == 0, 1.0, 0.0)
            x = jnp.concatenate([a.astype(jnp.bfloat16),
                                 ones_col.astype(jnp.bfloat16)], axis=1)
            out_ref[...] += jax.lax.dot_general(
                oh_t, x, (((1,), (0,)), ((), ())),
                preferred_element_type=jnp.float32)

    def run(synthon_feats, rg0_W1, rg0_b1, skey_W, skey_b, idx):
        idx = idx.astype(jnp.int32)
        idx3 = idx.reshape(NBI, 1, BI)
        bounds = jnp.searchsorted(idx, jnp.arange(0, NR + 1, BO,
                                                  dtype=jnp.int32))
        sb = jnp.minimum(bounds[:-1], S - 1) // BI
        eb = jnp.clip(bounds[1:] - 1, 0, S - 1) // BI
        eb = jnp.maximum(eb, sb)
        blk_start = sb.astype(jnp.int32)
        blk_len = (eb - sb + 1).astype(jnp.int32)

        def pin(j, t, start, length):
            return jnp.minimum(start[j] + t, start[j] + length[j] - 1)

        grid = (NR // BO, NBI)
        full = lambda shape: pl.BlockSpec(shape, lambda *a: (0,) * len(shape))
        return pl.pallas_call(
            body,
            grid_spec=pltpu.PrefetchScalarGridSpec(
                num_scalar_prefetch=2,
                grid=grid,
                in_specs=[
                    pl.BlockSpec((1, 1, BI),
                                 lambda j, t, s, l: (pin(j, t, s, l), 0, 0)),
                    pl.BlockSpec((BI, Q),
                                 lambda j, t, s, l: (pin(j, t, s, l), 0)),
                    full((Q, H)), full((H,)),
                    full((Q, SK)), full((SK,)),
                ],
                out_specs=[
                    pl.BlockSpec((BO, HA), lambda j, t, s, l: (j, 0)),
                    pl.BlockSpec((BI, SK),
                                 lambda j, t, s, l: (pin(j, t, s, l), 0)),
                ],
            ),
            out_shape=[
                jax.ShapeDtypeStruct((NR, HA), jnp.float32),
                jax.ShapeDtypeStruct((S, SK), jnp.bfloat16),
            ],
        )(blk_start, blk_len, idx3, synthon_feats, rg0_W1, rg0_b1,
          skey_W, skey_b)

    return run


def _k23_body(idx_ref, pw_ref, gw2_ref, gb2_ref, w1_ref, b1_ref,
              w2_ref, b2_ref, x1_ref, c1_ref,
              x2_ref, c2_ref, rw1_ref, rb1_ref, rw2_ref, rb2_ref,
              kw_ref, kb_ref, out_ref, b_scr):
    i = pl.program_id(0)
    pw = pw_ref[...]
    cnt = jnp.maximum(pw[:, H], 1.0)
    mean_a = pw[:, :H] / cnt[:, None]
    rp = _mm(mean_a, gw2_ref[...]) + gb2_ref[...]
    a = jax.nn.relu(_mm(rp, w1_ref[...]) + b1_ref[...])
    rf = _mm(a, w2_ref[...]) + b2_ref[...]
    b = jax.nn.relu(_mm(rf, x1_ref[...]) + c1_ref[...])
    blk = b.shape[0]
    ones_col = jnp.where(lax.broadcasted_iota(jnp.int32, (blk, 128), 1) == 0,
                         1.0, 0.0)
    b_scr[pl.ds(i * blk, blk), :] = jnp.concatenate(
        [b, ones_col], axis=1).astype(jnp.bfloat16)

    @pl.when(i == NR // blk - 1)
    def _():
        idx = idx_ref[0, 0, :]
        oh = (lax.broadcasted_iota(jnp.int32, (R, NR), 0)
              == idx[None, :]).astype(jnp.bfloat16)
        sw = jax.lax.dot_general(oh, b_scr[...], (((1,), (0,)), ((), ())),
                                 preferred_element_type=jnp.float32)
        cnt2 = sw[:, H]
        rp2 = _mm(sw[:, :H], x2_ref[...]) + cnt2[:, None] * c2_ref[...]
        a2 = jax.nn.relu(_mm(rp2, rw1_ref[...]) + rb1_ref[...])
        rf2 = _mm(a2, rw2_ref[...]) + rb2_ref[...]
        out_ref[...] = (_mm(rf2, kw_ref[...]) + kb_ref[...]).astype(jnp.bfloat16)


def _run_k23(pooled_wide, rgroup2reaction, rg0_W2, rg0_b2,
             rg1_W1, rg1_b1, rg1_W2, rg1_b2, rx0_W1, rx0_b1,
             rx0_W2, rx0_b2, rx1_W1, rx1_b1, rx1_W2, rx1_b2,
             rkey_W, rkey_b):
    BLK = 512
    grid = (NR // BLK,)
    full = lambda shape: pl.BlockSpec(shape, lambda i: (0,) * len(shape))
    idx3 = rgroup2reaction.astype(jnp.int32).reshape(1, 1, NR)
    return pl.pallas_call(
        _k23_body,
        grid=grid,
        in_specs=[
            full((1, 1, NR)),
            pl.BlockSpec((BLK, HA), lambda i: (i, 0)),
            full((H, H)), full((H,)),
            full((H, H)), full((H,)), full((H, Q)), full((Q,)),
            full((Q, H)), full((H,)),
            full((H, H)), full((H,)), full((H, H)), full((H,)),
            full((H, Q)), full((Q,)),
            full((Q, RK)), full((RK,)),
        ],
        out_specs=pl.BlockSpec((R, RK), lambda i: (0, 0)),
        out_shape=jax.ShapeDtypeStruct((R, RK), jnp.bfloat16),
        scratch_shapes=[pltpu.VMEM((NR, HA), jnp.bfloat16)],
    )(idx3, pooled_wide, rg0_W2, rg0_b2, rg1_W1, rg1_b1, rg1_W2, rg1_b2,
      rx0_W1, rx0_b1, rx0_W2, rx0_b2, rx1_W1, rx1_b1, rx1_W2, rx1_b2,
      rkey_W, rkey_b)


def _log_softmax(x):
    lse = jnp.log(jnp.sum(jnp.exp(x), axis=1, keepdims=True))
    return x - lse


def _k5_body(x_ref, pw1_ref, pb1_ref, pw2_ref, pb2_ref,
             rw1_ref, rb1_ref, rw2_ref, rb2_ref,
             sw1_ref, sb1_ref, sw2_ref, sb2_ref,
             rk_ref, sk_ref, out_ref):
    x = x_ref[...]
    a = jax.nn.relu(_mm(x, pw1_ref[...]) + pb1_ref[...])
    q = x + _mm(a, pw2_ref[...]) + pb2_ref[...]
    ar = jax.nn.relu(_mm(q, rw1_ref[...]) + rb1_ref[...])
    rqa = _mm(ar, rw2_ref[...]) + rb2_ref[...]
    asq = jax.nn.relu(_mm(q, sw1_ref[...]) + sb1_ref[...])
    sqa = _mm(asq, sw2_ref[...]) + sb2_ref[...]
    rl = _mm_t(rqa, rk_ref[...])
    sl = _mm_t(sqa, sk_ref[...])
    out_ref[:, :R] = _log_softmax(rl)
    out_ref[:, R:] = _log_softmax(sl)


def _run_k5(queries, proc_W1, proc_b1, proc_W2, proc_b2,
            rq_W1, rq_b1, rq_W2, rq_b2, sq_W1, sq_b1, sq_W2, sq_b2,
            reaction_keys, synthon_keys):
    BLK = 256
    grid = (B // BLK,)
    full = lambda shape: pl.BlockSpec(shape, lambda i: (0,) * len(shape))
    return pl.pallas_call(
        _k5_body,
        grid=grid,
        in_specs=[
            pl.BlockSpec((BLK, Q), lambda i: (i, 0)),
            full((Q, H)), full((H,)), full((H, Q)), full((Q,)),
            full((Q, H)), full((H,)), full((H, RK)), full((RK,)),
            full((Q, H)), full((H,)), full((H, SK)), full((SK,)),
            full((R, RK)), full((S, SK)),
        ],
        out_specs=pl.BlockSpec((BLK, R + S), lambda i: (i, 0)),
        out_shape=jax.ShapeDtypeStruct((B, R + S), jnp.float32),
    )(queries, proc_W1, proc_b1, proc_W2, proc_b2,
      rq_W1, rq_b1, rq_W2, rq_b2, sq_W1, sq_b1, sq_W2, sq_b2,
      reaction_keys, synthon_keys)


def kernel(queries, synthon_feats, synthon2rgroup, rgroup2reaction,
           proc_W1, proc_b1, proc_W2, proc_b2,
           rg0_W1, rg0_b1, rg0_W2, rg0_b2,
           rg1_W1, rg1_b1, rg1_W2, rg1_b2,
           rx0_W1, rx0_b1, rx0_W2, rx0_b2,
           rx1_W1, rx1_b1, rx1_W2, rx1_b2,
           rkey_W, rkey_b, skey_W, skey_b,
           rq_W1, rq_b1, rq_W2, rq_b2,
           sq_W1, sq_b1, sq_W2, sq_b2):
    pooled_wide, synthon_keys = _make_fused_segsum()(
        synthon_feats, rg0_W1, rg0_b1, skey_W, skey_b, synthon2rgroup)
    reaction_keys = _run_k23(pooled_wide, rgroup2reaction, rg0_W2, rg0_b2,
                             rg1_W1, rg1_b1, rg1_W2, rg1_b2, rx0_W1, rx0_b1,
                             rx0_W2, rx0_b2, rx1_W1, rx1_b1, rx1_W2, rx1_b2,
                             rkey_W, rkey_b)
    return _run_k5(queries, proc_W1, proc_b1, proc_W2, proc_b2,
                   rq_W1, rq_b1, rq_W2, rq_b2, sq_W1, sq_b1, sq_W2, sq_b2,
                   reaction_keys, synthon_keys)
